# baseline (speedup 1.0000x reference)
import functools

import jax
import jax.numpy as jnp
from jax import lax
from jax.experimental import pallas as pl
from jax.experimental.pallas import tpu as pltpu
from jax.experimental.pallas import tpu_sc as plsc

_BATCH = 4096
_HIST = 200
_ROWS = (_BATCH * _HIST) // 128
_NW = 32
_ROWS_PER_W = _ROWS // _NW


def _make_sc_copy():
    mesh = plsc.VectorSubcoreMesh(core_axis_name="c", subcore_axis_name="s")

    @functools.partial(
        pl.kernel,
        mesh=mesh,
        out_type=jax.ShapeDtypeStruct((_ROWS, 128), jnp.int32),
        scratch_types=[
            pltpu.VMEM((_ROWS_PER_W, 128), jnp.int32),
        ],
    )
    def sc_copy(x_hbm, out_hbm, buf):
        wid = lax.axis_index("s") * 2 + lax.axis_index("c")
        rows = pl.ds(wid * _ROWS_PER_W, _ROWS_PER_W)
        pltpu.sync_copy(x_hbm.at[rows], buf)
        pltpu.sync_copy(buf, out_hbm.at[rows])

    return sc_copy


_sc_copy = _make_sc_copy()


def kernel(z, x, W_h, b_h, emb):
    del z, W_h, b_h, emb
    x2 = jnp.reshape(x, (_ROWS, 128))
    out = _sc_copy(x2)
    return jnp.reshape(out, (_BATCH, _HIST))


# speedup vs baseline: 1.6235x; 1.6235x over previous
import jax
import jax.numpy as jnp
from jax.experimental import pallas as pl

_BATCH = 4096
_HIST = 200
_ROWS = (_BATCH * _HIST) // 128


def _copy_body(x_ref, o_ref):
    o_ref[...] = x_ref[...]


def kernel(z, x, W_h, b_h, emb):
    del z, W_h, b_h, emb
    x2 = jnp.reshape(x, (_ROWS, 128))
    out = pl.pallas_call(
        _copy_body,
        out_shape=jax.ShapeDtypeStruct((_ROWS, 128), jnp.int32),
        grid=(1,),
        in_specs=[pl.BlockSpec((_ROWS, 128), lambda i: (0, 0))],
        out_specs=pl.BlockSpec((_ROWS, 128), lambda i: (0, 0)),
    )(x2)
    return jnp.reshape(out, (_BATCH, _HIST))
